# Initial kernel scaffold; baseline (speedup 1.0000x reference)
import jax
import jax.numpy as jnp
from jax.experimental import pallas as pl


def kernel(sequence, duration, max_length, W1, b1, W2, b2):
    raise NotImplementedError("write your pallas kernel here")


# speedup vs baseline: 1014.3517x; 1014.3517x over previous
import functools

import jax
import jax.numpy as jnp
from jax import lax
from jax.experimental import pallas as pl
from jax.experimental.pallas import tpu as pltpu
from jax.experimental.pallas import tpu_sc as plsc

_B, _T, _D = 16, 512, 512
_L = 1600
_NC, _NS = 2, 16
_NW = _NC * _NS
_ROWS_PER_W = _B * _L // _NW
_CH = 80
_NCHUNK = _ROWS_PER_W // _CH


def _tc_body(maxlen_ref, b2_ref, seq_ref, dur_ref, w1_ref, b1_ref, w2_ref,
             led_ref, fidx_ref):
    b = pl.program_id(0)
    x = seq_ref[0]
    h = jnp.maximum(
        jnp.dot(x, w1_ref[...], preferred_element_type=jnp.float32)
        + b1_ref[...], 0.0)
    led = jnp.dot(h, w2_ref[...],
                  preferred_element_type=jnp.float32) + b2_ref[0]
    led_ref[0] = led

    durc = dur_ref[0]
    cum = durc
    sh = 1
    while sh < _T:
        cum = cum + jnp.concatenate(
            [jnp.zeros((sh, 1), jnp.int32), cum[:-sh]], axis=0)
        sh *= 2
    total = jnp.sum(durc)
    limit = jnp.minimum(total, maxlen_ref[0])
    lrow = lax.broadcasted_iota(jnp.int32, (1, _L), 1)
    idx = jnp.sum((cum <= lrow).astype(jnp.int32), axis=0,
                  keepdims=True)
    fidx = jnp.where(lrow < limit,
                     b * _T + jnp.minimum(idx, _T - 1),
                     _B * _T)
    fidx_ref[0] = fidx


def _tc_call(maxlen, b2, seq, dur3, w1, b1_2, w2):
    return pl.pallas_call(
        _tc_body,
        grid=(_B,),
        in_specs=[
            pl.BlockSpec(memory_space=pltpu.SMEM),
            pl.BlockSpec(memory_space=pltpu.SMEM),
            pl.BlockSpec((1, _T, _D), lambda b: (b, 0, 0)),
            pl.BlockSpec((1, _T, 1), lambda b: (b, 0, 0)),
            pl.BlockSpec((_D, _D), lambda b: (0, 0)),
            pl.BlockSpec((1, _D), lambda b: (0, 0)),
            pl.BlockSpec((_D, 1), lambda b: (0, 0)),
        ],
        out_specs=[
            pl.BlockSpec((1, _T, 1), lambda b: (b, 0, 0)),
            pl.BlockSpec((1, 1, _L), lambda b: (b, 0, 0)),
        ],
        out_shape=[
            jax.ShapeDtypeStruct((_B, _T, 1), jnp.float32),
            jax.ShapeDtypeStruct((_B, 1, _L), jnp.int32),
        ],
    )(maxlen, b2, seq, dur3, w1, b1_2, w2)


def _sc_gather(table, fidx_flat):
    mesh = plsc.VectorSubcoreMesh(core_axis_name="c", subcore_axis_name="s")

    @functools.partial(
        pl.kernel,
        out_type=jax.ShapeDtypeStruct((_B * _L, _D), jnp.float32),
        mesh=mesh,
        scratch_types=[
            pltpu.VMEM((_ROWS_PER_W,), jnp.int32),
            pltpu.VMEM((_CH, _D), jnp.float32),
            pltpu.VMEM((_CH, _D), jnp.float32),
            pltpu.SemaphoreType.DMA,
            pltpu.SemaphoreType.DMA,
        ],
    )
    def k(table_hbm, fidx_hbm, out_hbm, idx_v, buf0, buf1, sem0, sem1):
        wid = lax.axis_index("s") * _NC + lax.axis_index("c")
        base = wid * _ROWS_PER_W
        pltpu.sync_copy(fidx_hbm.at[pl.ds(base, _ROWS_PER_W)], idx_v)
        bufs = (buf0, buf1)
        sems = (sem0, sem1)
        handles = [None, None]
        handles[0] = pltpu.async_copy(
            table_hbm.at[idx_v.at[pl.ds(0, _CH)]], bufs[0], sems[0])
        for ch in range(_NCHUNK):
            nxt = ch + 1
            if nxt < _NCHUNK:
                handles[nxt % 2] = pltpu.async_copy(
                    table_hbm.at[idx_v.at[pl.ds(nxt * _CH, _CH)]],
                    bufs[nxt % 2], sems[nxt % 2])
            handles[ch % 2].wait()
            pltpu.sync_copy(bufs[ch % 2],
                            out_hbm.at[pl.ds(base + ch * _CH, _CH)])

    return k(table, fidx_flat)


def kernel(sequence, duration, max_length, W1, b1, W2, b2):
    maxlen = jnp.asarray(max_length, jnp.int32).reshape(1)
    b2_arr = jnp.asarray(b2, jnp.float32).reshape(1)
    dur3 = duration.astype(jnp.int32).reshape(_B, _T, 1)
    b1_2 = b1.reshape(1, _D)
    led3, fidx3 = _tc_call(maxlen, b2_arr, sequence, dur3, W1, b1_2, W2)
    table = jnp.concatenate(
        [sequence.reshape(_B * _T, _D), jnp.zeros((8, _D), jnp.float32)],
        axis=0)
    aligned = _sc_gather(table, fidx3.reshape(_B * _L))
    return aligned.reshape(_B, _L, _D), led3.reshape(_B, _T)


# speedup vs baseline: 3870.0891x; 3.8153x over previous
import functools

import jax
import jax.numpy as jnp
from jax import lax
from jax.experimental import pallas as pl
from jax.experimental.pallas import tpu as pltpu
from jax.experimental.pallas import tpu_sc as plsc

_B, _T, _D = 16, 512, 512
_L = 1600
_NC, _NS = 2, 16
_NW = _NC * _NS
_CH = 40
_W = 48
_NCB = _L // _CH
_NCHT = _B * _NCB
_KPT = _NCHT // _NW
_MF = 16


def _tc_idx_body(maxlen_ref, dur_ref, fidx_ref, meta_ref):
    b = pl.program_id(0)
    durc = dur_ref[0]
    cum = durc
    sh = 1
    while sh < _T:
        cum = cum + jnp.concatenate(
            [jnp.zeros((sh, 1), jnp.int32), cum[:-sh]], axis=0)
        sh *= 2
    total = jnp.sum(durc)
    limit = jnp.minimum(total, maxlen_ref[0])
    mvi = jnp.sum((cum <= limit - 1).astype(jnp.int32))
    cap = jnp.minimum(mvi, _T - 1)

    lrow = lax.broadcasted_iota(jnp.int32, (1, _L), 1)
    idx = jnp.sum((cum <= lrow).astype(jnp.int32), axis=0, keepdims=True)
    fidx_ref[0] = b * _T + jnp.minimum(idx, cap)

    l0 = lax.broadcasted_iota(jnp.int32, (1, _NCB), 1) * _CH
    s40 = jnp.sum((cum <= l0).astype(jnp.int32), axis=0, keepdims=True)
    e40 = jnp.sum((cum <= l0 + (_CH - 1)).astype(jnp.int32), axis=0,
                  keepdims=True)
    start_unc = b * _T + jnp.minimum(s40, cap)
    start = jnp.minimum(jnp.bitwise_and(start_unc, jnp.int32(-8)),
                        _B * _T - _W)
    end = b * _T + jnp.minimum(e40, cap)
    fast = (end - start <= _W - 1).astype(jnp.int32)
    nvalid = jnp.clip(limit - l0, 0, _CH)
    pad = jnp.zeros((_MF - 3, _NCB), jnp.int32)
    meta_ref[0] = jnp.concatenate(
        [start, nvalid, fast, pad], axis=0)


def _tc_idx_call(maxlen, dur3):
    return pl.pallas_call(
        _tc_idx_body,
        grid=(_B,),
        in_specs=[
            pl.BlockSpec(memory_space=pltpu.SMEM),
            pl.BlockSpec((1, _T, 1), lambda b: (b, 0, 0)),
        ],
        out_specs=[
            pl.BlockSpec((1, 1, _L), lambda b: (b, 0, 0)),
            pl.BlockSpec((1, _MF, _NCB), lambda b: (b, 0, 0)),
        ],
        out_shape=[
            jax.ShapeDtypeStruct((_B, 1, _L), jnp.int32),
            jax.ShapeDtypeStruct((_B, _MF, _NCB), jnp.int32),
        ],
    )(maxlen, dur3)


def _tc_mlp_body(b2_ref, seq_ref, w1_ref, b1_ref, w2r_ref, led_ref):
    x = seq_ref[0]
    h = jnp.maximum(
        jnp.dot(x, w1_ref[...], preferred_element_type=jnp.float32)
        + b1_ref[...], 0.0)
    led = jnp.sum(h * w2r_ref[...], axis=1, keepdims=True) + b2_ref[0]
    led_ref[0] = led


def _tc_mlp_call(b2, seq, w1, b1_2, w2r):
    return pl.pallas_call(
        _tc_mlp_body,
        grid=(_B,),
        in_specs=[
            pl.BlockSpec(memory_space=pltpu.SMEM),
            pl.BlockSpec((1, _T, _D), lambda b: (b, 0, 0)),
            pl.BlockSpec((_D, _D), lambda b: (0, 0)),
            pl.BlockSpec((1, _D), lambda b: (0, 0)),
            pl.BlockSpec((1, _D), lambda b: (0, 0)),
        ],
        out_specs=pl.BlockSpec((1, _T, 1), lambda b: (b, 0, 0)),
        out_shape=jax.ShapeDtypeStruct((_B, _T, 1), jnp.float32),
    )(b2, seq, w1, b1_2, w2r)


def _sc_expand(seq_flat, fidx_perm, meta_perm):
    mesh = plsc.VectorSubcoreMesh(core_axis_name="c", subcore_axis_name="s")

    @functools.partial(
        pl.kernel,
        out_type=jax.ShapeDtypeStruct((_B * _L, _D), jnp.float32),
        mesh=mesh,
        scratch_types=[
            pltpu.VMEM((_KPT, _CH), jnp.int32),
            pltpu.VMEM((_KPT, _MF), jnp.int32),
            pltpu.VMEM((_W, _D), jnp.float32),
            pltpu.VMEM((_W, _D), jnp.float32),
            pltpu.VMEM((_CH, _D), jnp.float32),
            pltpu.VMEM((_CH, _D), jnp.float32),
            pltpu.SemaphoreType.DMA,
            pltpu.SemaphoreType.DMA,
            pltpu.SemaphoreType.DMA,
            pltpu.SemaphoreType.DMA,
            pltpu.SemaphoreType.DMA,
        ],
    )
    def k(seq_hbm, fidx_hbm, meta_hbm, out_hbm, idx_v, meta_v,
          in0, in1, o0, o1, isem0, isem1, osem0, osem1, ssem):
        wid = lax.axis_index("s") * _NC + lax.axis_index("c")
        pltpu.sync_copy(fidx_hbm.at[wid], idx_v)
        pltpu.sync_copy(meta_hbm.at[wid], meta_v)
        ins = (in0, in1)
        outs = (o0, o1)
        isems = (isem0, isem1)
        osems = (osem0, osem1)
        zero16 = jnp.zeros((16,), jnp.float32)

        def meta_row(kk):
            return meta_v[kk, pl.ds(0, _MF)]

        def issue_inload(kk, p):
            mr = meta_row(kk)

            @pl.when((mr[1] > 0) & (mr[2] > 0))
            def _():
                pltpu.async_copy(
                    seq_hbm.at[pl.ds(pl.multiple_of(mr[0], 8), _W)],
                    ins[p], isems[p])

        def wait_store(p):
            pltpu.make_async_copy(outs[p], out_hbm.at[pl.ds(0, _CH)],
                                  osems[p]).wait()

        def proc(kk, p):
            mr = meta_row(kk)
            st = mr[0]
            nv = mr[1]
            fast = mr[2]

            @pl.when((nv > 0) & (fast > 0))
            def _fast():
                pltpu.make_async_copy(seq_hbm.at[pl.ds(0, _W)],
                                      ins[p], isems[p]).wait()
                for gs in (0, 16, 24):
                    svec = idx_v[kk, pl.ds(gs, 16)] - st
                    for lane in range(16):
                        r = gs + lane

                        @pl.when(r < nv)
                        def _row(r=r, s=svec[lane]):
                            for j in range(_D // 16):
                                outs[p][r, pl.ds(16 * j, 16)] = (
                                    ins[p][s, pl.ds(16 * j, 16)])

            @pl.when((nv > 0) & (fast == 0))
            def _slow():
                pltpu.async_copy(seq_hbm.at[idx_v.at[kk]], outs[p],
                                 ssem).wait()

            def zbody(r, carry):
                for j in range(_D // 16):
                    outs[p][r, pl.ds(16 * j, 16)] = zero16
                return carry
            lax.fori_loop(nv, _CH, zbody, 0)
            orow = (kk * _NW + wid) * _CH
            pltpu.async_copy(outs[p],
                             out_hbm.at[pl.ds(pl.multiple_of(orow, 8), _CH)],
                             osems[p])

        issue_inload(0, 0)

        def body2(m, carry):
            kk0 = 2 * m
            kk1 = kk0 + 1
            issue_inload(kk1, 1)

            @pl.when(m > 0)
            def _():
                wait_store(0)
            proc(kk0, 0)

            @pl.when(kk0 + 2 < _KPT)
            def _():
                issue_inload(kk0 + 2, 0)

            @pl.when(m > 0)
            def _():
                wait_store(1)
            proc(kk1, 1)
            return carry

        lax.fori_loop(0, _KPT // 2, body2, 0)
        wait_store(0)
        wait_store(1)

    return k(seq_flat, fidx_perm, meta_perm)


def kernel(sequence, duration, max_length, W1, b1, W2, b2):
    maxlen = jnp.asarray(max_length, jnp.int32).reshape(1)
    b2_arr = jnp.asarray(b2, jnp.float32).reshape(1)
    dur3 = duration.astype(jnp.int32).reshape(_B, _T, 1)
    fidx3, meta3 = _tc_idx_call(maxlen, dur3)
    fidx_perm = (fidx3.reshape(_NCHT // _NW, _NW, _CH)
                 .transpose(1, 0, 2))
    meta_perm = (meta3.transpose(0, 2, 1).reshape(_NCHT // _NW, _NW, _MF)
                 .transpose(1, 0, 2))
    aligned = _sc_expand(sequence.reshape(_B * _T, _D), fidx_perm, meta_perm)
    led3 = _tc_mlp_call(b2_arr, sequence, W1, b1.reshape(1, _D),
                        W2.reshape(1, _D))
    return aligned.reshape(_B, _L, _D), led3.reshape(_B, _T)


# speedup vs baseline: 4546.2536x; 1.1747x over previous
import functools

import jax
import jax.numpy as jnp
from jax import lax
from jax.experimental import pallas as pl
from jax.experimental.pallas import tpu as pltpu
from jax.experimental.pallas import tpu_sc as plsc

_B, _T, _D = 16, 512, 512
_L = 1600
_NC, _NS = 2, 16
_NW = _NC * _NS
_CH = 40
_W = 48
_NCB = _L // _CH
_NCHT = _B * _NCB
_KPT = _NCHT // _NW
_MF = 16


def _tc_idx_body(maxlen_ref, dur_ref, fidx_ref, meta_ref):
    b = pl.program_id(0)
    durc = dur_ref[0]
    cum = durc
    sh = 1
    while sh < _T:
        cum = cum + jnp.concatenate(
            [jnp.zeros((sh, 1), jnp.int32), cum[:-sh]], axis=0)
        sh *= 2
    total = jnp.sum(durc)
    limit = jnp.minimum(total, maxlen_ref[0])
    mvi = jnp.sum((cum <= limit - 1).astype(jnp.int32))
    cap = jnp.minimum(mvi, _T - 1)

    lrow = lax.broadcasted_iota(jnp.int32, (1, _L), 1)
    idx = jnp.sum((cum <= lrow).astype(jnp.int32), axis=0, keepdims=True)
    fidx_ref[0] = b * _T + jnp.minimum(idx, cap)

    l0 = lax.broadcasted_iota(jnp.int32, (1, _NCB), 1) * _CH
    s40 = jnp.sum((cum <= l0).astype(jnp.int32), axis=0, keepdims=True)
    e40 = jnp.sum((cum <= l0 + (_CH - 1)).astype(jnp.int32), axis=0,
                  keepdims=True)
    start_unc = b * _T + jnp.minimum(s40, cap)
    start = jnp.minimum(jnp.bitwise_and(start_unc, jnp.int32(-8)),
                        _B * _T - _W)
    end = b * _T + jnp.minimum(e40, cap)
    fast = (end - start <= _W - 1).astype(jnp.int32)
    nvalid = jnp.clip(limit - l0, 0, _CH)
    pad = jnp.zeros((_MF - 3, _NCB), jnp.int32)
    meta_ref[0] = jnp.concatenate(
        [start, nvalid, fast, pad], axis=0)


def _tc_idx_call(maxlen, dur3):
    return pl.pallas_call(
        _tc_idx_body,
        grid=(_B,),
        in_specs=[
            pl.BlockSpec(memory_space=pltpu.SMEM),
            pl.BlockSpec((1, _T, 1), lambda b: (b, 0, 0)),
        ],
        out_specs=[
            pl.BlockSpec((1, 1, _L), lambda b: (b, 0, 0)),
            pl.BlockSpec((1, _MF, _NCB), lambda b: (b, 0, 0)),
        ],
        out_shape=[
            jax.ShapeDtypeStruct((_B, 1, _L), jnp.int32),
            jax.ShapeDtypeStruct((_B, _MF, _NCB), jnp.int32),
        ],
    )(maxlen, dur3)


def _tc_mlp_body(b2_ref, seq_ref, w1_ref, b1_ref, w2r_ref, led_ref):
    x = seq_ref[0]
    h = jnp.maximum(
        jnp.dot(x, w1_ref[...], preferred_element_type=jnp.float32)
        + b1_ref[...], 0.0)
    led = jnp.sum(h * w2r_ref[...], axis=1, keepdims=True) + b2_ref[0]
    led_ref[0] = led


def _tc_mlp_call(b2, seq, w1, b1_2, w2r):
    return pl.pallas_call(
        _tc_mlp_body,
        grid=(_B,),
        in_specs=[
            pl.BlockSpec(memory_space=pltpu.SMEM),
            pl.BlockSpec((1, _T, _D), lambda b: (b, 0, 0)),
            pl.BlockSpec((_D, _D), lambda b: (0, 0)),
            pl.BlockSpec((1, _D), lambda b: (0, 0)),
            pl.BlockSpec((1, _D), lambda b: (0, 0)),
        ],
        out_specs=pl.BlockSpec((1, _T, 1), lambda b: (b, 0, 0)),
        out_shape=jax.ShapeDtypeStruct((_B, _T, 1), jnp.float32),
    )(b2, seq, w1, b1_2, w2r)


def _sc_expand(seq_flat, fidx_perm, meta_perm):
    mesh = plsc.VectorSubcoreMesh(core_axis_name="c", subcore_axis_name="s")

    @functools.partial(
        pl.kernel,
        out_type=jax.ShapeDtypeStruct((_B * _L, _D), jnp.float32),
        mesh=mesh,
        scratch_types=[
            pltpu.VMEM((_KPT, _CH), jnp.int32),
            pltpu.VMEM((_KPT, _MF), jnp.int32),
            pltpu.VMEM((_W, _D), jnp.float32),
            pltpu.VMEM((_W, _D), jnp.float32),
            pltpu.VMEM((_CH, _D), jnp.float32),
            pltpu.VMEM((_CH, _D), jnp.float32),
            pltpu.SemaphoreType.DMA,
            pltpu.SemaphoreType.DMA,
            pltpu.SemaphoreType.DMA,
            pltpu.SemaphoreType.DMA,
            pltpu.SemaphoreType.DMA,
        ],
    )
    def k(seq_hbm, fidx_hbm, meta_hbm, out_hbm, idx_v, meta_v,
          in0, in1, o0, o1, isem0, isem1, osem0, osem1, ssem):
        wid = lax.axis_index("s") * _NC + lax.axis_index("c")
        pltpu.sync_copy(fidx_hbm.at[wid], idx_v)
        pltpu.sync_copy(meta_hbm.at[wid], meta_v)
        ins = (in0, in1)
        outs = (o0, o1)
        isems = (isem0, isem1)
        osems = (osem0, osem1)
        zero16 = jnp.zeros((16,), jnp.float32)

        def meta_row(kk):
            return meta_v[kk, pl.ds(0, _MF)]

        def issue_inload(kk, p):
            mr = meta_row(kk)

            @pl.when((mr[1] > 0) & (mr[2] > 0))
            def _():
                pltpu.async_copy(
                    seq_hbm.at[pl.ds(pl.multiple_of(mr[0], 8), _W)],
                    ins[p], isems[p])

        def wait_store(p):
            pltpu.make_async_copy(outs[p], out_hbm.at[pl.ds(0, _CH)],
                                  osems[p]).wait()

        def proc(kk, p):
            mr = meta_row(kk)
            st = mr[0]
            nv = mr[1]
            fast = mr[2]

            @pl.when((nv > 0) & (fast > 0))
            def _fast():
                pltpu.make_async_copy(seq_hbm.at[pl.ds(0, _W)],
                                      ins[p], isems[p]).wait()
                for gs in (0, 16, 24):
                    svec = idx_v[kk, pl.ds(gs, 16)] - st
                    for lane in range(16):
                        r = gs + lane
                        s = svec[lane]
                        vals = [ins[p][s, pl.ds(16 * j, 16)]
                                for j in range(_D // 16)]
                        for j in range(_D // 16):
                            outs[p][r, pl.ds(16 * j, 16)] = vals[j]

            @pl.when((nv > 0) & (fast == 0))
            def _slow():
                pltpu.async_copy(seq_hbm.at[idx_v.at[kk]], outs[p],
                                 ssem).wait()

            def zbody(r, carry):
                for j in range(_D // 16):
                    outs[p][r, pl.ds(16 * j, 16)] = zero16
                return carry
            lax.fori_loop(nv, _CH, zbody, 0)
            orow = (kk * _NW + wid) * _CH
            pltpu.async_copy(outs[p],
                             out_hbm.at[pl.ds(pl.multiple_of(orow, 8), _CH)],
                             osems[p])

        issue_inload(0, 0)

        def body2(m, carry):
            kk0 = 2 * m
            kk1 = kk0 + 1
            issue_inload(kk1, 1)

            @pl.when(m > 0)
            def _():
                wait_store(0)
            proc(kk0, 0)

            @pl.when(kk0 + 2 < _KPT)
            def _():
                issue_inload(kk0 + 2, 0)

            @pl.when(m > 0)
            def _():
                wait_store(1)
            proc(kk1, 1)
            return carry

        lax.fori_loop(0, _KPT // 2, body2, 0)
        wait_store(0)
        wait_store(1)

    return k(seq_flat, fidx_perm, meta_perm)


def kernel(sequence, duration, max_length, W1, b1, W2, b2):
    maxlen = jnp.asarray(max_length, jnp.int32).reshape(1)
    b2_arr = jnp.asarray(b2, jnp.float32).reshape(1)
    dur3 = duration.astype(jnp.int32).reshape(_B, _T, 1)
    fidx3, meta3 = _tc_idx_call(maxlen, dur3)
    fidx_perm = (fidx3.reshape(_NCHT // _NW, _NW, _CH)
                 .transpose(1, 0, 2))
    meta_perm = (meta3.transpose(0, 2, 1).reshape(_NCHT // _NW, _NW, _MF)
                 .transpose(1, 0, 2))
    aligned = _sc_expand(sequence.reshape(_B * _T, _D), fidx_perm, meta_perm)
    led3 = _tc_mlp_call(b2_arr, sequence, W1, b1.reshape(1, _D),
                        W2.reshape(1, _D))
    return aligned.reshape(_B, _L, _D), led3.reshape(_B, _T)


# speedup vs baseline: 7181.3525x; 1.5796x over previous
import functools

import jax
import jax.numpy as jnp
from jax import lax
from jax.experimental import pallas as pl
from jax.experimental.pallas import tpu as pltpu
from jax.experimental.pallas import tpu_sc as plsc

_B, _T, _D = 16, 512, 512
_L = 1600
_NC, _NS = 2, 16
_NW = _NC * _NS
_CH = 40
_W = 48
_NCB = _L // _CH
_NCHT = _B * _NCB
_KPT = _NCHT // _NW
_MF = 16


def _tc_idx_body(maxlen_ref, dur_ref, fidx_ref, meta_ref):
    b = pl.program_id(0)
    durc = dur_ref[0]
    cum = durc
    sh = 1
    while sh < _T:
        cum = cum + jnp.concatenate(
            [jnp.zeros((sh, 1), jnp.int32), cum[:-sh]], axis=0)
        sh *= 2
    total = jnp.sum(durc)
    limit = jnp.minimum(total, maxlen_ref[0])
    mvi = jnp.sum((cum <= limit - 1).astype(jnp.int32))
    cap = jnp.minimum(mvi, _T - 1)

    lrow = lax.broadcasted_iota(jnp.int32, (1, _L), 1)
    idx = jnp.sum((cum <= lrow).astype(jnp.int32), axis=0, keepdims=True)
    fidx_ref[0] = b * _T + jnp.minimum(idx, cap)

    l0 = lax.broadcasted_iota(jnp.int32, (1, _NCB), 1) * _CH
    s40 = jnp.sum((cum <= l0).astype(jnp.int32), axis=0, keepdims=True)
    e40 = jnp.sum((cum <= l0 + (_CH - 1)).astype(jnp.int32), axis=0,
                  keepdims=True)
    start_unc = b * _T + jnp.minimum(s40, cap)
    start = jnp.minimum(jnp.bitwise_and(start_unc, jnp.int32(-8)),
                        _B * _T - _W)
    end = b * _T + jnp.minimum(e40, cap)
    fast = (end - start <= _W - 1).astype(jnp.int32)
    nvalid = jnp.clip(limit - l0, 0, _CH)
    pad = jnp.zeros((_MF - 3, _NCB), jnp.int32)
    meta_ref[0] = jnp.concatenate(
        [start, nvalid, fast, pad], axis=0)


def _tc_idx_call(maxlen, dur3):
    return pl.pallas_call(
        _tc_idx_body,
        grid=(_B,),
        in_specs=[
            pl.BlockSpec(memory_space=pltpu.SMEM),
            pl.BlockSpec((1, _T, 1), lambda b: (b, 0, 0)),
        ],
        out_specs=[
            pl.BlockSpec((1, 1, _L), lambda b: (b, 0, 0)),
            pl.BlockSpec((1, _MF, _NCB), lambda b: (b, 0, 0)),
        ],
        out_shape=[
            jax.ShapeDtypeStruct((_B, 1, _L), jnp.int32),
            jax.ShapeDtypeStruct((_B, _MF, _NCB), jnp.int32),
        ],
    )(maxlen, dur3)


def _tc_mlp_body(b2_ref, seq_ref, w1_ref, b1_ref, w2r_ref, led_ref):
    x = seq_ref[0]
    h = jnp.maximum(
        jnp.dot(x, w1_ref[...], preferred_element_type=jnp.float32)
        + b1_ref[...], 0.0)
    led = jnp.sum(h * w2r_ref[...], axis=1, keepdims=True) + b2_ref[0]
    led_ref[0] = led


def _tc_mlp_call(b2, seq, w1, b1_2, w2r):
    return pl.pallas_call(
        _tc_mlp_body,
        grid=(_B,),
        in_specs=[
            pl.BlockSpec(memory_space=pltpu.SMEM),
            pl.BlockSpec((1, _T, _D), lambda b: (b, 0, 0)),
            pl.BlockSpec((_D, _D), lambda b: (0, 0)),
            pl.BlockSpec((1, _D), lambda b: (0, 0)),
            pl.BlockSpec((1, _D), lambda b: (0, 0)),
        ],
        out_specs=pl.BlockSpec((1, _T, 1), lambda b: (b, 0, 0)),
        out_shape=jax.ShapeDtypeStruct((_B, _T, 1), jnp.float32),
    )(b2, seq, w1, b1_2, w2r)


def _sc_expand(seq_flat, fidx_perm, meta_perm):
    mesh = plsc.VectorSubcoreMesh(core_axis_name="c", subcore_axis_name="s")

    @functools.partial(
        pl.kernel,
        out_type=jax.ShapeDtypeStruct((_B * _L, _D), jnp.float32),
        mesh=mesh,
        scratch_types=[
            pltpu.VMEM((_KPT, _CH), jnp.int32),
            pltpu.VMEM((_KPT, _MF), jnp.int32),
            pltpu.VMEM((_W, _D), jnp.float32),
            pltpu.VMEM((_W, _D), jnp.float32),
            pltpu.VMEM((_CH, _D), jnp.float32),
            pltpu.VMEM((_CH, _D), jnp.float32),
            pltpu.VMEM((_CH, _D), jnp.float32),
            pltpu.SemaphoreType.DMA,
            pltpu.SemaphoreType.DMA,
            pltpu.SemaphoreType.DMA,
            pltpu.SemaphoreType.DMA,
            pltpu.SemaphoreType.DMA,
        ],
    )
    def k(seq_hbm, fidx_hbm, meta_hbm, out_hbm, idx_v, meta_v,
          in0, in1, o0, o1, zbuf, isem0, isem1, osem0, osem1, ssem):
        wid = lax.axis_index("s") * _NC + lax.axis_index("c")
        pltpu.sync_copy(fidx_hbm.at[wid], idx_v)
        pltpu.sync_copy(meta_hbm.at[wid], meta_v)
        ins = (in0, in1)
        outs = (o0, o1)
        isems = (isem0, isem1)
        osems = (osem0, osem1)
        zero16 = jnp.zeros((16,), jnp.float32)

        def zinit(r, carry):
            for j in range(_D // 16):
                zbuf[r, pl.ds(16 * j, 16)] = zero16
            return carry
        lax.fori_loop(0, _CH, zinit, 0)

        def meta_row(kk):
            return meta_v[kk, pl.ds(0, _MF)]

        def issue_inload(kk, p):
            mr = meta_row(kk)

            @pl.when((mr[1] > 0) & (mr[2] > 0))
            def _():
                pltpu.async_copy(
                    seq_hbm.at[pl.ds(pl.multiple_of(mr[0], 8), _W)],
                    ins[p], isems[p])

        def wait_store(p):
            pltpu.make_async_copy(outs[p], out_hbm.at[pl.ds(0, _CH)],
                                  osems[p]).wait()

        def proc(kk, p):
            mr = meta_row(kk)
            st = mr[0]
            nv = mr[1]
            fast = mr[2]
            orow = (kk * _NW + wid) * _CH
            odst = out_hbm.at[pl.ds(pl.multiple_of(orow, 8), _CH)]

            @pl.when(nv == 0)
            def _allpad():
                pltpu.async_copy(zbuf, odst, osems[p])

            @pl.when(nv > 0)
            def _some():
                @pl.when(fast > 0)
                def _fast():
                    pltpu.make_async_copy(seq_hbm.at[pl.ds(0, _W)],
                                          ins[p], isems[p]).wait()
                    rows = []
                    for gs, lanes in ((0, range(16)), (16, range(16)),
                                      (24, range(8, 16))):
                        svec = idx_v[kk, pl.ds(gs, 16)] - st
                        for lane in lanes:
                            rows.append((gs + lane, svec[lane]))
                    nj = _D // 16
                    vals = [ins[p][rows[0][1], pl.ds(16 * j, 16)]
                            for j in range(nj)]
                    for i in range(1, _CH):
                        r_prev = rows[i - 1][0]
                        s_cur = rows[i][1]
                        nvals = []
                        for j in range(nj):
                            outs[p][r_prev, pl.ds(16 * j, 16)] = vals[j]
                            nvals.append(ins[p][s_cur, pl.ds(16 * j, 16)])
                        vals = nvals
                    for j in range(nj):
                        outs[p][rows[-1][0], pl.ds(16 * j, 16)] = vals[j]

                @pl.when(fast == 0)
                def _slow():
                    pltpu.async_copy(seq_hbm.at[idx_v.at[kk]], outs[p],
                                     ssem).wait()

                def zbody(r, carry):
                    for j in range(_D // 16):
                        outs[p][r, pl.ds(16 * j, 16)] = zero16
                    return carry
                lax.fori_loop(nv, _CH, zbody, 0)
                pltpu.async_copy(outs[p], odst, osems[p])

        issue_inload(0, 0)

        def body2(m, carry):
            kk0 = 2 * m
            kk1 = kk0 + 1
            issue_inload(kk1, 1)

            @pl.when(m > 0)
            def _():
                wait_store(0)
            proc(kk0, 0)

            @pl.when(kk0 + 2 < _KPT)
            def _():
                issue_inload(kk0 + 2, 0)

            @pl.when(m > 0)
            def _():
                wait_store(1)
            proc(kk1, 1)
            return carry

        lax.fori_loop(0, _KPT // 2, body2, 0)
        wait_store(0)
        wait_store(1)

    return k(seq_flat, fidx_perm, meta_perm)


def kernel(sequence, duration, max_length, W1, b1, W2, b2):
    maxlen = jnp.asarray(max_length, jnp.int32).reshape(1)
    b2_arr = jnp.asarray(b2, jnp.float32).reshape(1)
    dur3 = duration.astype(jnp.int32).reshape(_B, _T, 1)
    fidx3, meta3 = _tc_idx_call(maxlen, dur3)
    fidx_perm = (fidx3.reshape(_NCHT // _NW, _NW, _CH)
                 .transpose(1, 0, 2))
    meta_perm = (meta3.transpose(0, 2, 1).reshape(_NCHT // _NW, _NW, _MF)
                 .transpose(1, 0, 2))
    aligned = _sc_expand(sequence.reshape(_B * _T, _D), fidx_perm, meta_perm)
    led3 = _tc_mlp_call(b2_arr, sequence, W1, b1.reshape(1, _D),
                        W2.reshape(1, _D))
    return aligned.reshape(_B, _L, _D), led3.reshape(_B, _T)
